# baseline (speedup 1.0000x reference)
import functools

import jax
import jax.numpy as jnp
from jax import lax
from jax.experimental import pallas as pl
from jax.experimental.pallas import tpu as pltpu
from jax.experimental.pallas import tpu_sc as plsc

N = 10000
E = 320000
C = 128

NC = 2
NS = 16
NW = NC * NS
EW = E // NW
NH = N // NC
NR = NH + 8
DUMP = NH

BE = 128
PAD = 3 * BE
CAP = 10368
CAPB = CAP + PAD
CAPCH = CAP // BE

BED = 80
NCHD = EW // BED
NPAD = 10240
DPT = NPAD // NS

ZB = 80
TPT = 320

BLK = 1000
GRID = N // BLK

_sc_mesh = plsc.VectorSubcoreMesh(core_axis_name="c", subcore_axis_name="s")


@functools.partial(
    pl.kernel,
    out_type=jax.ShapeDtypeStruct((NW * NPAD,), jnp.float32),
    mesh=_sc_mesh,
    scratch_types=[
        pltpu.VMEM((NCHD, BED), jnp.int32),
        pltpu.VMEM((NPAD,), jnp.float32),
    ],
    compiler_params=pltpu.CompilerParams(needs_layout_passes=False),
)
def _sc_degree(edge_hbm, out_hbm, idx_v, hist_v):
    cid = lax.axis_index("c")
    sid = lax.axis_index("s")
    wid = cid * NS + sid

    zero16 = jnp.zeros((16,), jnp.float32)

    def zbody(i, carry):
        hist_v[pl.ds(i * 16, 16)] = zero16
        return carry

    lax.fori_loop(0, NPAD // 16, zbody, 0)

    pltpu.sync_copy(edge_hbm.at[1, wid], idx_v)

    ones16 = jnp.ones((16,), jnp.float32)
    vpr = BED // 16

    def cbody(i, carry):
        idx = idx_v[i // vpr, pl.ds((i % vpr) * 16, 16)]
        plsc.addupdate_scatter(hist_v, [idx], ones16)
        return carry

    lax.fori_loop(0, EW // 16, cbody, 0)
    pltpu.sync_copy(hist_v, out_hbm.at[pl.ds(wid * NPAD, NPAD)])


@functools.partial(
    pl.kernel,
    out_type=[
        jax.ShapeDtypeStruct((2 * 2 * NW * CAP,), jnp.int32),
        jax.ShapeDtypeStruct((NW * 2 * 128,), jnp.int32),
    ],
    mesh=_sc_mesh,
    scratch_types=[
        pltpu.VMEM((NCHD, BED), jnp.int32),
        pltpu.VMEM((NCHD, BED), jnp.int32),
        pltpu.VMEM((CAPB,), jnp.int32),
        pltpu.VMEM((CAPB,), jnp.int32),
        pltpu.VMEM((CAPB,), jnp.int32),
        pltpu.VMEM((CAPB,), jnp.int32),
        pltpu.VMEM((128,), jnp.int32),
    ],
    compiler_params=pltpu.CompilerParams(needs_layout_passes=False),
)
def _sc_part(edge_hbm, lists_hbm, counts_hbm,
             srcv, dstv, bAs, bAd, bBs, bBd, cnt_v):
    cid = lax.axis_index("c")
    sid = lax.axis_index("s")
    wid = cid * NS + sid

    pltpu.sync_copy(edge_hbm.at[0, wid], srcv)
    pltpu.sync_copy(edge_hbm.at[1, wid], dstv)

    vpr = BED // 16
    nh16 = jnp.full((16,), NH, jnp.int32)
    iota16 = lax.iota(jnp.int32, 16)

    def it(i, carry):
        cA, cB = carry
        r = i // vpr
        cl = (i % vpr) * 16
        s = srcv[r, pl.ds(cl, 16)]
        d = dstv[r, pl.ds(cl, 16)]
        mA = d < nh16
        mB = jnp.logical_not(mA)
        mAi = mA.astype(jnp.int32)
        exA = plsc.cumsum(mAi) - mAi
        exB = iota16 - exA
        idxA = exA + cA
        idxB = exB + cB
        plsc.store_scatter(bAs, [idxA], s, mask=mA)
        plsc.store_scatter(bAd, [idxA], d, mask=mA)
        plsc.store_scatter(bBs, [idxB], s, mask=mB)
        plsc.store_scatter(bBd, [idxB], d - nh16, mask=mB)
        nA = jnp.sum(mAi)
        return (cA + nA, cB + (16 - nA))

    cA, cB = lax.fori_loop(0, EW // 16, it,
                           (jnp.int32(0), jnp.int32(0)))

    zero16i = jnp.zeros((16,), jnp.int32)
    dump16 = jnp.full((16,), DUMP, jnp.int32)

    def fill(k, carry):
        off = iota16 + k * 16
        plsc.store_scatter(bAs, [off + cA], zero16i)
        plsc.store_scatter(bAd, [off + cA], dump16)
        plsc.store_scatter(bBs, [off + cB], zero16i)
        plsc.store_scatter(bBd, [off + cB], dump16)
        return carry

    lax.fori_loop(0, PAD // 16, fill, 0)

    padA = ((jnp.maximum(cA, 1) + PAD - 1) // PAD) * PAD
    padB = ((jnp.maximum(cB, 1) + PAD - 1) // PAD) * PAD

    cnt_v[pl.ds(0, 16)] = zero16i + padA
    pltpu.sync_copy(cnt_v, counts_hbm.at[pl.ds((wid * 2 + 0) * 128, 128)])
    cnt_v[pl.ds(0, 16)] = zero16i + padB
    pltpu.sync_copy(cnt_v, counts_hbm.at[pl.ds((wid * 2 + 1) * 128, 128)])

    def seg(io, h):
        return pl.ds(((io * 2 + h) * NW + wid) * CAP, CAP)

    pltpu.sync_copy(bAs.at[pl.ds(0, CAP)], lists_hbm.at[seg(0, 0)])
    pltpu.sync_copy(bAd.at[pl.ds(0, CAP)], lists_hbm.at[seg(1, 0)])
    pltpu.sync_copy(bBs.at[pl.ds(0, CAP)], lists_hbm.at[seg(0, 1)])
    pltpu.sync_copy(bBd.at[pl.ds(0, CAP)], lists_hbm.at[seg(1, 1)])


@functools.partial(
    pl.kernel,
    out_type=jax.ShapeDtypeStruct((N, C), jnp.float32),
    mesh=_sc_mesh,
    scratch_types=[
        pltpu.VMEM((CAPCH, BE), jnp.int32),
        pltpu.VMEM((CAPCH, BE), jnp.int32),
        pltpu.VMEM((3, BE, C), jnp.float32),
        pltpu.VMEM((ZB, C), jnp.float32),
        pltpu.VMEM((128,), jnp.int32),
        pltpu.VMEM_SHARED((NR, C), jnp.float32),
        [pltpu.SemaphoreType.DMA] * 3,
        [pltpu.SemaphoreType.DMA] * 3,
    ],
    compiler_params=pltpu.CompilerParams(needs_layout_passes=False),
)
def _sc_edges(lists_hbm, counts_hbm, hp_hbm, out_hbm,
              sidx_v, didx_v, rows_v, zb_v, cnt_v, acc_s, gsem, ssem):
    cid = lax.axis_index("c")
    sid = lax.axis_index("s")
    base = sid * TPT

    zero16 = jnp.zeros((16,), jnp.float32)

    def zbody(r, carry):
        for c in range(C // 16):
            zb_v[r, pl.ds(c * 16, 16)] = zero16
        return carry

    lax.fori_loop(0, ZB, zbody, 0)

    @pl.when(sid < NS - 1)
    def _():
        for k in range(4):
            pltpu.sync_copy(zb_v, acc_s.at[pl.ds(base + k * ZB, ZB)])

    @pl.when(sid == NS - 1)
    def _():
        pltpu.sync_copy(zb_v, acc_s.at[pl.ds(base, ZB)])
        pltpu.sync_copy(zb_v, acc_s.at[pl.ds(base + ZB, ZB)])
        pltpu.sync_copy(zb_v.at[pl.ds(0, 48)],
                        acc_s.at[pl.ds(base + 2 * ZB, 48)])

    def gat(c, b):
        pltpu.async_copy(hp_hbm.at[sidx_v.at[c]], rows_v.at[b], gsem[b])

    def gat_wait(c, b):
        pltpu.make_async_copy(hp_hbm.at[sidx_v.at[c]], rows_v.at[b],
                              gsem[b]).wait()

    def sca(c, b):
        pltpu.async_copy(rows_v.at[b], acc_s.at[didx_v.at[c]], ssem[b],
                         add=True)

    def sca_wait(c, b):
        pltpu.make_async_copy(rows_v.at[b], acc_s.at[didx_v.at[c]],
                              ssem[b]).wait()

    plsc.subcore_barrier()

    for j in range(2):
        wsrc = sid * 2 + j
        pltpu.sync_copy(lists_hbm.at[0, cid, wsrc], sidx_v)
        pltpu.sync_copy(lists_hbm.at[1, cid, wsrc], didx_v)
        pltpu.sync_copy(
            counts_hbm.at[pl.ds((wsrc * 2 + cid) * 128, 128)], cnt_v)
        nch = cnt_v[pl.ds(0, 16)][0] // BE

        gat(0, 0)
        gat(1, 1)

        def pbody(p, carry):
            for b in range(3):
                c = 3 * p + b
                gat_wait(c, b)
                sca(c, b)
                if b == 0:
                    @pl.when(p > 0)
                    def _():
                        sca_wait(c - 1, 2)
                else:
                    sca_wait(c - 1, b - 1)

                @pl.when(c + 2 < nch)
                def _():
                    gat(c + 2, (b + 2) % 3)
            return carry

        lax.fori_loop(0, nch // 3, pbody, 0)
        sca_wait(nch - 1, 2)

    plsc.subcore_barrier()
    obase = cid * NH + base

    @pl.when(sid < NS - 1)
    def _():
        for k in range(4):
            pltpu.sync_copy(acc_s.at[pl.ds(base + k * ZB, ZB)], zb_v)
            pltpu.sync_copy(zb_v, out_hbm.at[pl.ds(obase + k * ZB, ZB)])

    @pl.when(sid == NS - 1)
    def _():
        for k in range(2):
            pltpu.sync_copy(acc_s.at[pl.ds(base + k * ZB, ZB)], zb_v)
            pltpu.sync_copy(zb_v, out_hbm.at[pl.ds(obase + k * ZB, ZB)])
        pltpu.sync_copy(acc_s.at[pl.ds(base + 2 * ZB, 40)],
                        zb_v.at[pl.ds(0, 40)])
        pltpu.sync_copy(zb_v.at[pl.ds(0, 40)],
                        out_hbm.at[pl.ds(obase + 2 * ZB, 40)])


def _kd_body(deg_ref, o_ref):
    d = deg_ref[...]
    o_ref[...] = lax.rsqrt(jnp.sum(d, axis=0, keepdims=True) + 1.0)


_kd = pl.pallas_call(
    _kd_body,
    out_shape=jax.ShapeDtypeStruct((1, NPAD), jnp.float32),
)


def _ka_body(x_ref, w_ref, dinv_ref, o_ref):
    h = jnp.dot(x_ref[...], w_ref[...], preferred_element_type=jnp.float32)
    o_ref[...] = h * dinv_ref[...]


_ka = pl.pallas_call(
    _ka_body,
    grid=(GRID,),
    in_specs=[
        pl.BlockSpec((BLK, C), lambda i: (i, 0)),
        pl.BlockSpec((C, C), lambda i: (0, 0)),
        pl.BlockSpec((BLK, 1), lambda i: (i, 0)),
    ],
    out_specs=pl.BlockSpec((BLK, C), lambda i: (i, 0)),
    out_shape=jax.ShapeDtypeStruct((N, C), jnp.float32),
)


def _kb_body(acc_ref, hp_ref, dinv_ref, b_ref, y_ref, s1_ref, s2_ref):
    i = pl.program_id(0)
    y = (acc_ref[...] + hp_ref[...]) * dinv_ref[...] + b_ref[...]
    y_ref[...] = y
    ps1 = jnp.sum(y, axis=0, keepdims=True)
    ps2 = jnp.sum(y * y, axis=0, keepdims=True)

    @pl.when(i == 0)
    def _():
        s1_ref[...] = ps1
        s2_ref[...] = ps2

    @pl.when(i > 0)
    def _():
        s1_ref[...] += ps1
        s2_ref[...] += ps2


_kb = pl.pallas_call(
    _kb_body,
    grid=(GRID,),
    in_specs=[
        pl.BlockSpec((BLK, C), lambda i: (i, 0)),
        pl.BlockSpec((BLK, C), lambda i: (i, 0)),
        pl.BlockSpec((BLK, 1), lambda i: (i, 0)),
        pl.BlockSpec((1, C), lambda i: (0, 0)),
    ],
    out_specs=[
        pl.BlockSpec((BLK, C), lambda i: (i, 0)),
        pl.BlockSpec((1, C), lambda i: (0, 0)),
        pl.BlockSpec((1, C), lambda i: (0, 0)),
    ],
    out_shape=[
        jax.ShapeDtypeStruct((N, C), jnp.float32),
        jax.ShapeDtypeStruct((1, C), jnp.float32),
        jax.ShapeDtypeStruct((1, C), jnp.float32),
    ],
)


def _kc_body(with_res, y_ref, s1_ref, s2_ref, gw_ref, gb_ref, ga_ref,
             *rest):
    if with_res:
        res_ref, o_ref = rest
    else:
        (o_ref,) = rest
    ga = ga_ref[...]
    mu = s1_ref[...] * (1.0 / N)
    var = s2_ref[...] * (1.0 / N) - mu * mu * ga * (2.0 - ga)
    xn = gw_ref[...] * (y_ref[...] - ga * mu) * lax.rsqrt(var + 1e-5)
    out = jnp.maximum(xn + gb_ref[...], 0.0)
    if with_res:
        out = out + res_ref[...]
    o_ref[...] = out


def _make_kc(with_res):
    vec = pl.BlockSpec((1, C), lambda i: (0, 0))
    big = pl.BlockSpec((BLK, C), lambda i: (i, 0))
    in_specs = [big, vec, vec, vec, vec, vec] + ([big] if with_res else [])
    return pl.pallas_call(
        functools.partial(_kc_body, with_res),
        grid=(GRID,),
        in_specs=in_specs,
        out_specs=big,
        out_shape=jax.ShapeDtypeStruct((N, C), jnp.float32),
    )


_kc0 = _make_kc(False)
_kc1 = _make_kc(True)


def _ke_body(x_ref, x1_ref, x2_ref, x3_ref, wh_ref, bh_ref, o_ref):
    acc = jnp.dot(x_ref[...], wh_ref[0:C], preferred_element_type=jnp.float32)
    acc += jnp.dot(x1_ref[...], wh_ref[C:2 * C],
                   preferred_element_type=jnp.float32)
    acc += jnp.dot(x2_ref[...], wh_ref[2 * C:3 * C],
                   preferred_element_type=jnp.float32)
    acc += jnp.dot(x3_ref[...], wh_ref[3 * C:4 * C],
                   preferred_element_type=jnp.float32)
    o_ref[...] = acc + bh_ref[...]


_ke = pl.pallas_call(
    _ke_body,
    grid=(GRID,),
    in_specs=[
        pl.BlockSpec((BLK, C), lambda i: (i, 0)),
        pl.BlockSpec((BLK, C), lambda i: (i, 0)),
        pl.BlockSpec((BLK, C), lambda i: (i, 0)),
        pl.BlockSpec((BLK, C), lambda i: (i, 0)),
        pl.BlockSpec((4 * C, C), lambda i: (0, 0)),
        pl.BlockSpec((1, C), lambda i: (0, 0)),
    ],
    out_specs=pl.BlockSpec((BLK, C), lambda i: (i, 0)),
    out_shape=jax.ShapeDtypeStruct((N, C), jnp.float32),
)


def _conv(xk, lists5, counts, dinv, W, b, gw, gb, ga, res):
    hp = _ka(xk, W, dinv)
    acc = _sc_edges(lists5, counts, hp)
    y, s1, s2 = _kb(acc, hp, dinv, b)
    if res is None:
        return _kc0(y, s1, s2, gw, gb, ga)
    return _kc1(y, s1, s2, gw, gb, ga, res)


def kernel(x, edge_index, W1, b1, g1w, g1b, g1a, W2, b2, g2w, g2b, g2a,
           W3, b3, g3w, g3b, g3a, Wh, bh):
    e4d = edge_index.reshape(2, NW, NCHD, BED)
    row = lambda v: v.reshape(1, C)

    deg2 = _sc_degree(e4d).reshape(NW, NPAD)
    dinv = _kd(deg2).reshape(NPAD, 1)[:N]
    lists_flat, counts = _sc_part(e4d)
    lists5 = lists_flat.reshape(2, 2, NW, CAPCH, BE)

    x1 = _conv(x, lists5, counts, dinv, W1, row(b1), row(g1w), row(g1b),
               row(g1a), None)
    x2 = _conv(x1, lists5, counts, dinv, W2, row(b2), row(g2w), row(g2b),
               row(g2a), x1)
    x3 = _conv(x2, lists5, counts, dinv, W3, row(b3), row(g3w), row(g3b),
               row(g3a), x2)
    return _ke(x, x1, x2, x3, Wh, row(bh))


# speedup vs baseline: 3.3637x; 3.3637x over previous
import functools

import jax
import jax.numpy as jnp
from jax import lax
from jax.experimental import pallas as pl
from jax.experimental.pallas import tpu as pltpu
from jax.experimental.pallas import tpu_sc as plsc

N = 10000
E = 320000
C = 128
CC = 64

NC = 2
NS = 16
NW = NC * NS
EW = E // NW
BE = 125
NCH = EW // BE
BED = 80
NCHD = EW // BED
CH = 200
NCHK = N // CH
NPAD = 10240
DPT = NPAD // NS

BLK = 1000
GRID = N // BLK

_sc_mesh = plsc.VectorSubcoreMesh(core_axis_name="c", subcore_axis_name="s")


@functools.partial(
    pl.kernel,
    out_type=jax.ShapeDtypeStruct((NC * NPAD,), jnp.float32),
    mesh=_sc_mesh,
    scratch_types=[
        pltpu.VMEM((NCHD, BED), jnp.int32),
        pltpu.VMEM((NPAD,), jnp.float32),
        pltpu.VMEM((NS * DPT,), jnp.float32),
        pltpu.VMEM((DPT,), jnp.float32),
        pltpu.VMEM_SHARED((NS * NPAD,), jnp.float32),
    ],
    compiler_params=pltpu.CompilerParams(needs_layout_passes=False),
)
def _sc_degree(edge_hbm, out_hbm, idx_v, hist_v, col_v, red_v, stage_s):
    cid = lax.axis_index("c")
    sid = lax.axis_index("s")
    wid = cid * NS + sid

    zero16 = jnp.zeros((16,), jnp.float32)

    def zbody(i, carry):
        hist_v[pl.ds(i * 16, 16)] = zero16
        return carry

    lax.fori_loop(0, NPAD // 16, zbody, 0)

    pltpu.sync_copy(edge_hbm.at[1, wid], idx_v)

    ones16 = jnp.ones((16,), jnp.float32)
    vpr = BED // 16

    def cbody(i, carry):
        idx = idx_v[i // vpr, pl.ds((i % vpr) * 16, 16)]
        plsc.addupdate_scatter(hist_v, [idx], ones16)
        return carry

    lax.fori_loop(0, EW // 16, cbody, 0)

    pltpu.sync_copy(hist_v, stage_s.at[pl.ds(sid * NPAD, NPAD)])
    plsc.subcore_barrier()
    for r in range(NS):
        pltpu.sync_copy(stage_s.at[pl.ds(r * NPAD + sid * DPT, DPT)],
                        col_v.at[pl.ds(r * DPT, DPT)])

    def rbody(k, carry):
        acc = col_v[pl.ds(k * 16, 16)]
        for r in range(1, NS):
            acc = acc + col_v[pl.ds(r * DPT + k * 16, 16)]
        red_v[pl.ds(k * 16, 16)] = acc
        return carry

    lax.fori_loop(0, DPT // 16, rbody, 0)
    pltpu.sync_copy(red_v, out_hbm.at[pl.ds(cid * NPAD + sid * DPT, DPT)])


@functools.partial(
    pl.kernel,
    out_type=[
        jax.ShapeDtypeStruct((NC, N, CC), jnp.float32),
        jax.ShapeDtypeStruct((NC, N, CC), jnp.float32),
    ],
    mesh=_sc_mesh,
    scratch_types=[
        pltpu.VMEM((NCH, BE), jnp.int32),
        pltpu.VMEM((NCH, BE), jnp.int32),
        pltpu.VMEM((4, BE, CC), jnp.float32),
        pltpu.VMEM((CH, CC), jnp.float32),
        pltpu.VMEM_SHARED((N, CC), jnp.float32),
        [pltpu.SemaphoreType.DMA] * 4,
        [pltpu.SemaphoreType.DMA] * 4,
    ],
    compiler_params=pltpu.CompilerParams(needs_layout_passes=False,
                                         use_tc_tiling_on_sc=False),
)
def _sc_conv(edge_hbm, hp0_hbm, hp1_hbm, outa_hbm, outb_hbm,
             sidx_v, didx_v, rows_v, zb_v, acc_s, gsem, ssem):
    cid = lax.axis_index("c")
    sid = lax.axis_index("s")
    wid = cid * NS + sid
    ck0 = sid * 3 + jnp.minimum(sid, 2)
    ckn = jnp.where(sid < 2, 4, 3)

    zero16 = jnp.zeros((16,), jnp.float32)

    def zbody(r, carry):
        for c in range(CC // 16):
            zb_v[r, pl.ds(c * 16, 16)] = zero16
        return carry

    lax.fori_loop(0, CH, zbody, 0)

    pltpu.sync_copy(edge_hbm.at[0, wid], sidx_v)
    pltpu.sync_copy(edge_hbm.at[1, wid], didx_v)

    def gat(c, b, hp_hbm):
        pltpu.async_copy(hp_hbm.at[sidx_v.at[c]], rows_v.at[b], gsem[b])

    def gat_wait(c, b, hp_hbm):
        pltpu.make_async_copy(hp_hbm.at[sidx_v.at[c]], rows_v.at[b],
                              gsem[b]).wait()

    def sca(c, b):
        pltpu.async_copy(rows_v.at[b], acc_s.at[didx_v.at[c]], ssem[b],
                         add=True)

    def sca_wait(c, b):
        pltpu.make_async_copy(rows_v.at[b], acc_s.at[didx_v.at[c]],
                              ssem[b]).wait()

    for hp_hbm, out_hbm in ((hp0_hbm, outa_hbm), (hp1_hbm, outb_hbm)):
        for b in range(3):
            gat(b, b, hp_hbm)

        def zcopy(k, carry):
            pltpu.sync_copy(zb_v, acc_s.at[pl.ds((ck0 + k) * CH, CH)])
            return carry

        lax.fori_loop(0, ckn, zcopy, 0)
        plsc.subcore_barrier()

        def pbody(p, carry):
            for b in range(4):
                c = 4 * p + b
                gat_wait(c, b, hp_hbm)
                sca(c, b)
                if b == 0:
                    @pl.when(p > 0)
                    def _():
                        sca_wait(c - 1, 3)
                else:
                    sca_wait(c - 1, b - 1)
                gat(c + 3, (b + 3) % 4, hp_hbm)
            return carry

        lax.fori_loop(0, (NCH - 4) // 4, pbody, 0)
        gat_wait(NCH - 4, 0, hp_hbm)
        sca(NCH - 4, 0)
        sca_wait(NCH - 5, 3)
        gat(NCH - 1, 3, hp_hbm)
        for b in range(1, 4):
            c = NCH - 4 + b
            gat_wait(c, b, hp_hbm)
            sca(c, b)
        for b in range(4):
            sca_wait(NCH - 4 + b, b)

        plsc.subcore_barrier()

        def wcopy(k, carry):
            sl = pl.ds((ck0 + k) * CH, CH)
            pltpu.sync_copy(acc_s.at[sl], zb_v)
            pltpu.sync_copy(zb_v, out_hbm.at[cid, sl])
            return carry

        lax.fori_loop(0, ckn, wcopy, 0)
        lax.fori_loop(0, CH, zbody, 0)


def _kd_body(deg_ref, o_ref):
    d = deg_ref[...]
    o_ref[...] = lax.rsqrt(d[0:1, :] + d[1:2, :] + 1.0)


_kd = pl.pallas_call(
    _kd_body,
    out_shape=jax.ShapeDtypeStruct((1, NPAD), jnp.float32),
)


def _ka_body(x_ref, w_ref, dinv_ref, o0_ref, o1_ref):
    h = jnp.dot(x_ref[...], w_ref[...], preferred_element_type=jnp.float32)
    hp = h * dinv_ref[...]
    o0_ref[...] = hp[:, :CC]
    o1_ref[...] = hp[:, CC:]


_ka = pl.pallas_call(
    _ka_body,
    grid=(GRID,),
    in_specs=[
        pl.BlockSpec((BLK, C), lambda i: (i, 0)),
        pl.BlockSpec((C, C), lambda i: (0, 0)),
        pl.BlockSpec((BLK, 1), lambda i: (i, 0)),
    ],
    out_specs=[
        pl.BlockSpec((BLK, CC), lambda i: (i, 0)),
        pl.BlockSpec((BLK, CC), lambda i: (i, 0)),
    ],
    out_shape=[
        jax.ShapeDtypeStruct((N, CC), jnp.float32),
        jax.ShapeDtypeStruct((N, CC), jnp.float32),
    ],
)


def _kb_body(acca_ref, accb_ref, hp0_ref, hp1_ref, dinv_ref, b_ref,
             y_ref, s1_ref, s2_ref):
    i = pl.program_id(0)
    a = acca_ref[...]
    b = accb_ref[...]
    dinv = dinv_ref[...]
    y0 = (a[0] + a[1] + hp0_ref[...]) * dinv
    y1 = (b[0] + b[1] + hp1_ref[...]) * dinv
    y = jnp.concatenate([y0, y1], axis=1) + b_ref[...]
    y_ref[...] = y
    ps1 = jnp.sum(y, axis=0, keepdims=True)
    ps2 = jnp.sum(y * y, axis=0, keepdims=True)

    @pl.when(i == 0)
    def _():
        s1_ref[...] = ps1
        s2_ref[...] = ps2

    @pl.when(i > 0)
    def _():
        s1_ref[...] += ps1
        s2_ref[...] += ps2


_kb = pl.pallas_call(
    _kb_body,
    grid=(GRID,),
    in_specs=[
        pl.BlockSpec((NC, BLK, CC), lambda i: (0, i, 0)),
        pl.BlockSpec((NC, BLK, CC), lambda i: (0, i, 0)),
        pl.BlockSpec((BLK, CC), lambda i: (i, 0)),
        pl.BlockSpec((BLK, CC), lambda i: (i, 0)),
        pl.BlockSpec((BLK, 1), lambda i: (i, 0)),
        pl.BlockSpec((1, C), lambda i: (0, 0)),
    ],
    out_specs=[
        pl.BlockSpec((BLK, C), lambda i: (i, 0)),
        pl.BlockSpec((1, C), lambda i: (0, 0)),
        pl.BlockSpec((1, C), lambda i: (0, 0)),
    ],
    out_shape=[
        jax.ShapeDtypeStruct((N, C), jnp.float32),
        jax.ShapeDtypeStruct((1, C), jnp.float32),
        jax.ShapeDtypeStruct((1, C), jnp.float32),
    ],
)


def _kc_body(with_res, y_ref, s1_ref, s2_ref, gw_ref, gb_ref, ga_ref,
             *rest):
    if with_res:
        res_ref, o_ref = rest
    else:
        (o_ref,) = rest
    ga = ga_ref[...]
    mu = s1_ref[...] * (1.0 / N)
    var = s2_ref[...] * (1.0 / N) - mu * mu * ga * (2.0 - ga)
    xn = gw_ref[...] * (y_ref[...] - ga * mu) * lax.rsqrt(var + 1e-5)
    out = jnp.maximum(xn + gb_ref[...], 0.0)
    if with_res:
        out = out + res_ref[...]
    o_ref[...] = out


def _make_kc(with_res):
    vec = pl.BlockSpec((1, C), lambda i: (0, 0))
    big = pl.BlockSpec((BLK, C), lambda i: (i, 0))
    in_specs = [big, vec, vec, vec, vec, vec] + ([big] if with_res else [])
    return pl.pallas_call(
        functools.partial(_kc_body, with_res),
        grid=(GRID,),
        in_specs=in_specs,
        out_specs=big,
        out_shape=jax.ShapeDtypeStruct((N, C), jnp.float32),
    )


_kc0 = _make_kc(False)
_kc1 = _make_kc(True)


def _ke_body(x_ref, x1_ref, x2_ref, x3_ref, wh_ref, bh_ref, o_ref):
    acc = jnp.dot(x_ref[...], wh_ref[0:C], preferred_element_type=jnp.float32)
    acc += jnp.dot(x1_ref[...], wh_ref[C:2 * C],
                   preferred_element_type=jnp.float32)
    acc += jnp.dot(x2_ref[...], wh_ref[2 * C:3 * C],
                   preferred_element_type=jnp.float32)
    acc += jnp.dot(x3_ref[...], wh_ref[3 * C:4 * C],
                   preferred_element_type=jnp.float32)
    o_ref[...] = acc + bh_ref[...]


_ke = pl.pallas_call(
    _ke_body,
    grid=(GRID,),
    in_specs=[
        pl.BlockSpec((BLK, C), lambda i: (i, 0)),
        pl.BlockSpec((BLK, C), lambda i: (i, 0)),
        pl.BlockSpec((BLK, C), lambda i: (i, 0)),
        pl.BlockSpec((BLK, C), lambda i: (i, 0)),
        pl.BlockSpec((4 * C, C), lambda i: (0, 0)),
        pl.BlockSpec((1, C), lambda i: (0, 0)),
    ],
    out_specs=pl.BlockSpec((BLK, C), lambda i: (i, 0)),
    out_shape=jax.ShapeDtypeStruct((N, C), jnp.float32),
)


def _conv(xk, e4, dinv, W, b, gw, gb, ga, res):
    hp0, hp1 = _ka(xk, W, dinv)
    acca, accb = _sc_conv(e4, hp0, hp1)
    y, s1, s2 = _kb(acca, accb, hp0, hp1, dinv, b)
    if res is None:
        return _kc0(y, s1, s2, gw, gb, ga)
    return _kc1(y, s1, s2, gw, gb, ga, res)


def kernel(x, edge_index, W1, b1, g1w, g1b, g1a, W2, b2, g2w, g2b, g2a,
           W3, b3, g3w, g3b, g3a, Wh, bh):
    e4 = edge_index.reshape(2, NW, NCH, BE)
    e4d = edge_index.reshape(2, NW, NCHD, BED)
    row = lambda v: v.reshape(1, C)

    deg2 = _sc_degree(e4d).reshape(NC, NPAD)
    dinv = _kd(deg2).reshape(NPAD, 1)[:N]

    x1 = _conv(x, e4, dinv, W1, row(b1), row(g1w), row(g1b), row(g1a), None)
    x2 = _conv(x1, e4, dinv, W2, row(b2), row(g2w), row(g2b), row(g2a), x1)
    x3 = _conv(x2, e4, dinv, W3, row(b3), row(g3w), row(g3b), row(g3a), x2)
    return _ke(x, x1, x2, x3, Wh, row(bh))
